# baseline (device time: 78530 ns/iter reference)
import jax
import jax.numpy as jnp
from jax import lax
from jax.experimental import pallas as pl
from jax.experimental.pallas import tpu as pltpu

N_DEV = 4
B, Sq, D = 2, 256, 768
Hq, Dh = 8, 64
Dq = Hq * Dh
R = B * Sq
PACK_COLS = Dq + 2 * Hq
SCALE = 0.125


def kernel(x, Wq, Wo, K_ext, V_ext):
    Skv = K_ext.shape[1]

    xf = x.reshape(R, D)
    Kf = K_ext.reshape(B * Skv, Dq)
    Vf = V_ext.reshape(B * Skv, Dq)

    def body(x_ref, wq_ref, wo_ref, k_ref, v_ref, out_ref,
             comm_ref, send_sems, recv_sems):
        my = lax.axis_index("i")
        left = lax.rem(my + N_DEV - 1, N_DEV)
        right = lax.rem(my + 1, N_DEV)

        barrier = pltpu.get_barrier_semaphore()
        pl.semaphore_signal(barrier, inc=1, device_id=(left,),
                            device_id_type=pl.DeviceIdType.MESH)
        pl.semaphore_signal(barrier, inc=1, device_id=(right,),
                            device_id_type=pl.DeviceIdType.MESH)
        pl.semaphore_wait(barrier, 2)

        Q = jnp.dot(x_ref[:], wq_ref[:], preferred_element_type=jnp.float32)

        o_acc = [[None] * Hq for _ in range(B)]
        m_acc = [[None] * Hq for _ in range(B)]
        l_acc = [[None] * Hq for _ in range(B)]
        for b in range(B):
            for h in range(Hq):
                q_bh = Q[b * Sq:(b + 1) * Sq, h * Dh:(h + 1) * Dh]
                k_bh = k_ref[b * Skv:(b + 1) * Skv, h * Dh:(h + 1) * Dh]
                v_bh = v_ref[b * Skv:(b + 1) * Skv, h * Dh:(h + 1) * Dh]
                s = lax.dot_general(
                    q_bh, k_bh, (((1,), (1,)), ((), ())),
                    preferred_element_type=jnp.float32) * SCALE
                m_bh = jnp.max(s, axis=1, keepdims=True)
                p = jnp.exp(s - m_bh)
                l_bh = jnp.sum(p, axis=1, keepdims=True)
                o_bh = jnp.dot(p, v_bh,
                               preferred_element_type=jnp.float32)
                o_acc[b][h], m_acc[b][h], l_acc[b][h] = o_bh, m_bh, l_bh
                r0 = b * Sq
                comm_ref[0, r0:r0 + Sq, h * Dh:(h + 1) * Dh] = o_bh
                comm_ref[0, r0:r0 + Sq, Dq + h:Dq + h + 1] = m_bh
                comm_ref[0, r0:r0 + Sq, Dq + Hq + h:Dq + Hq + h + 1] = l_bh

        for hop in range(N_DEV - 1):
            rdma = pltpu.make_async_remote_copy(
                src_ref=comm_ref.at[hop],
                dst_ref=comm_ref.at[hop + 1],
                send_sem=send_sems.at[hop],
                recv_sem=recv_sems.at[hop],
                device_id=(right,),
                device_id_type=pl.DeviceIdType.MESH,
            )
            rdma.start()
            rdma.wait()

            recv = comm_ref[hop + 1]
            for b in range(B):
                for h in range(Hq):
                    r0 = b * Sq
                    o_r = recv[r0:r0 + Sq, h * Dh:(h + 1) * Dh]
                    m_r = recv[r0:r0 + Sq, Dq + h:Dq + h + 1]
                    l_r = recv[r0:r0 + Sq, Dq + Hq + h:Dq + Hq + h + 1]
                    m_new = jnp.maximum(m_acc[b][h], m_r)
                    a_old = jnp.exp(m_acc[b][h] - m_new)
                    a_new = jnp.exp(m_r - m_new)
                    l_acc[b][h] = l_acc[b][h] * a_old + l_r * a_new
                    o_acc[b][h] = o_acc[b][h] * a_old + o_r * a_new
                    m_acc[b][h] = m_new

        att = jnp.concatenate(
            [jnp.concatenate(
                [o_acc[b][h] / l_acc[b][h] for h in range(Hq)], axis=1)
             for b in range(B)],
            axis=0,
        )
        out_ref[:] = jnp.dot(att, wo_ref[:],
                             preferred_element_type=jnp.float32)

    out2d = pl.pallas_call(
        body,
        out_shape=jax.ShapeDtypeStruct((R, D), jnp.float32),
        in_specs=[pl.BlockSpec(memory_space=pltpu.VMEM)] * 5,
        out_specs=pl.BlockSpec(memory_space=pltpu.VMEM),
        scratch_shapes=[
            pltpu.VMEM((N_DEV, R, PACK_COLS), jnp.float32),
            pltpu.SemaphoreType.DMA((N_DEV - 1,)),
            pltpu.SemaphoreType.DMA((N_DEV - 1,)),
        ],
        compiler_params=pltpu.CompilerParams(collective_id=0),
    )(xf, Wq, Wo, Kf, Vf)

    return out2d.reshape(B, Sq, D)


# device time: 35227 ns/iter; 2.2293x vs baseline; 2.2293x over previous
import jax
import jax.numpy as jnp
from jax import lax
from jax.experimental import pallas as pl
from jax.experimental.pallas import tpu as pltpu

N_DEV = 4
B, Sq, D = 2, 256, 768
Hq, Dh = 8, 64
Dq = Hq * Dh
R = B * Sq
GROUP_ROWS = Dq + 2 * Hq
SCALE = 0.125
COMM_DTYPE = jnp.bfloat16


def kernel(x, Wq, Wo, K_ext, V_ext):
    Skv = K_ext.shape[1]

    xf = x.reshape(R, D)
    Kf = K_ext.reshape(B * Skv, Dq)
    Vf = V_ext.reshape(B * Skv, Dq)

    def body(x_ref, wq_ref, wo_ref, k_ref, v_ref, out_ref,
             cw_ref, ccw_ref, cw_send, cw_recv, ccw_send, ccw_recv):
        my = lax.axis_index("i")
        left = lax.rem(my + N_DEV - 1, N_DEV)
        right = lax.rem(my + 1, N_DEV)

        barrier = pltpu.get_barrier_semaphore()
        pl.semaphore_signal(barrier, inc=1, device_id=(left,),
                            device_id_type=pl.DeviceIdType.MESH)
        pl.semaphore_signal(barrier, inc=1, device_id=(right,),
                            device_id_type=pl.DeviceIdType.MESH)
        pl.semaphore_wait(barrier, 2)

        Q = jnp.dot(x_ref[:], wq_ref[:], preferred_element_type=jnp.float32)

        o_acc = [[None] * Hq for _ in range(B)]
        m_acc = [[None] * Hq for _ in range(B)]
        l_acc = [[None] * Hq for _ in range(B)]

        def compute_group(b, buf_ref):
            for h in range(Hq):
                q_bh = Q[b * Sq:(b + 1) * Sq, h * Dh:(h + 1) * Dh]
                k_bh = k_ref[b * Skv:(b + 1) * Skv, h * Dh:(h + 1) * Dh]
                v_bh = v_ref[b * Skv:(b + 1) * Skv, h * Dh:(h + 1) * Dh]
                sT = lax.dot_general(
                    k_bh, q_bh, (((1,), (1,)), ((), ())),
                    preferred_element_type=jnp.float32) * SCALE
                m_row = jnp.max(sT, axis=0, keepdims=True)
                m_row = m_row.astype(COMM_DTYPE).astype(jnp.float32)
                pT = jnp.exp(sT - m_row)
                l_row = jnp.sum(pT, axis=0, keepdims=True)
                oT = lax.dot_general(
                    v_bh, pT, (((0,), (0,)), ((), ())),
                    preferred_element_type=jnp.float32)
                o_acc[b][h], m_acc[b][h], l_acc[b][h] = oT, m_row, l_row
                buf_ref[0, h * Dh:(h + 1) * Dh, :] = oT.astype(COMM_DTYPE)
                buf_ref[0, Dq + h:Dq + h + 1, :] = m_row.astype(COMM_DTYPE)
                buf_ref[0, Dq + Hq + h:Dq + Hq + h + 1, :] = (
                    l_row.astype(COMM_DTYPE))

        def make_rdma(buf_ref, hop, send_sems, recv_sems, target):
            return pltpu.make_async_remote_copy(
                src_ref=buf_ref.at[hop],
                dst_ref=buf_ref.at[hop + 1],
                send_sem=send_sems.at[hop],
                recv_sem=recv_sems.at[hop],
                device_id=(target,),
                device_id_type=pl.DeviceIdType.MESH,
            )

        def merge_group(b, buf_ref, slot):
            recv = buf_ref[slot].astype(jnp.float32)
            for h in range(Hq):
                o_r = recv[h * Dh:(h + 1) * Dh, :]
                m_r = recv[Dq + h:Dq + h + 1, :]
                l_r = recv[Dq + Hq + h:Dq + Hq + h + 1, :]
                m_new = jnp.maximum(m_acc[b][h], m_r)
                a_old = jnp.exp(m_acc[b][h] - m_new)
                a_new = jnp.exp(m_r - m_new)
                l_acc[b][h] = l_acc[b][h] * a_old + l_r * a_new
                o_acc[b][h] = o_acc[b][h] * a_old + o_r * a_new
                m_acc[b][h] = m_new

        descs = []
        compute_group(0, cw_ref)
        cw_d = [make_rdma(cw_ref, 0, cw_send, cw_recv, right)]
        cw_d[0].start()
        compute_group(1, ccw_ref)
        ccw_d = [make_rdma(ccw_ref, 0, ccw_send, ccw_recv, left)]
        ccw_d[0].start()
        descs += [cw_d[0], ccw_d[0]]

        for hop in range(N_DEV - 1):
            cw_d[hop].wait_recv()
            ccw_d[hop].wait_recv()
            if hop < N_DEV - 2:
                dn = make_rdma(cw_ref, hop + 1, cw_send, cw_recv, right)
                dn.start()
                cw_d.append(dn)
                dn = make_rdma(ccw_ref, hop + 1, ccw_send, ccw_recv, left)
                dn.start()
                ccw_d.append(dn)
                descs += [cw_d[-1], ccw_d[-1]]
            merge_group(0, cw_ref, hop + 1)
            merge_group(1, ccw_ref, hop + 1)

        for d_ in descs:
            d_.wait_send()

        attT = jnp.concatenate(
            [jnp.concatenate(
                [o_acc[b][h] / l_acc[b][h] for h in range(Hq)], axis=0)
             for b in range(B)],
            axis=1,
        )
        out_ref[:] = lax.dot_general(
            attT, wo_ref[:], (((0,), (0,)), ((), ())),
            preferred_element_type=jnp.float32)

    out2d = pl.pallas_call(
        body,
        out_shape=jax.ShapeDtypeStruct((R, D), jnp.float32),
        in_specs=[pl.BlockSpec(memory_space=pltpu.VMEM)] * 5,
        out_specs=pl.BlockSpec(memory_space=pltpu.VMEM),
        scratch_shapes=[
            pltpu.VMEM((N_DEV, GROUP_ROWS, Sq), COMM_DTYPE),
            pltpu.VMEM((N_DEV, GROUP_ROWS, Sq), COMM_DTYPE),
            pltpu.SemaphoreType.DMA((N_DEV - 1,)),
            pltpu.SemaphoreType.DMA((N_DEV - 1,)),
            pltpu.SemaphoreType.DMA((N_DEV - 1,)),
            pltpu.SemaphoreType.DMA((N_DEV - 1,)),
        ],
        compiler_params=pltpu.CompilerParams(collective_id=0),
    )(xf, Wq, Wo, Kf, Vf)

    return out2d.reshape(B, Sq, D)
